# baseline (speedup 1.0000x reference)
import functools

import jax
import jax.numpy as jnp
from jax import lax
from jax.experimental import pallas as pl
from jax.experimental.pallas import tpu as pltpu
from jax.experimental.pallas import tpu_sc as plsc

_NUM_CORES = 2
_NUM_SUBCORES = 16
_NW = _NUM_CORES * _NUM_SUBCORES
_CHUNK = 128
_HALF = _CHUNK // 2
_NBUF = 4


@functools.lru_cache(maxsize=None)
def _build(n_rows, dim):
    rows_per_w = n_rows // _NW
    chunks_per_w = rows_per_w // _CHUNK
    n_groups = chunks_per_w // _NBUF
    mesh = plsc.VectorSubcoreMesh(core_axis_name="c", subcore_axis_name="s")

    @functools.partial(
        pl.kernel,
        mesh=mesh,
        out_type=jax.ShapeDtypeStruct((n_rows // 2, 2 * dim), jnp.float32),
        scratch_types=(
            [pltpu.VMEM((chunks_per_w, _CHUNK), jnp.int32)]
            + [pltpu.VMEM((2, _HALF, dim), jnp.float32) for _ in range(_NBUF)]
            + [pltpu.SemaphoreType.DMA for _ in range(2 * _NBUF)]
        ),
        compiler_params=pltpu.CompilerParams(
            use_tc_tiling_on_sc=False, skip_device_barrier=True
        ),
    )
    def run(idx_hbm, table_hbm, out_hbm, idx_v, *bufs_and_sems):
        bufs = bufs_and_sems[:_NBUF]
        gsems = bufs_and_sems[_NBUF : 2 * _NBUF]
        osems = bufs_and_sems[2 * _NBUF :]
        wid = lax.axis_index("s") * _NUM_CORES + lax.axis_index("c")
        pltpu.sync_copy(idx_hbm.at[pl.ds(wid * chunks_per_w, chunks_per_w)], idx_v)
        pbase = wid * (rows_per_w // 2)

        def gathers(j, b):
            left = pltpu.make_async_copy(
                table_hbm.at[idx_v.at[j, pl.ds(0, _HALF)]],
                bufs[b].at[0],
                gsems[b],
            )
            right = pltpu.make_async_copy(
                table_hbm.at[idx_v.at[j, pl.ds(_HALF, _HALF)]],
                bufs[b].at[1],
                gsems[b],
            )
            return left, right

        def stores(j, b):
            p = pl.ds(pbase + j * _HALF, _HALF)
            left = pltpu.make_async_copy(
                bufs[b].at[0], out_hbm.at[p, pl.ds(0, dim)], osems[b]
            )
            right = pltpu.make_async_copy(
                bufs[b].at[1], out_hbm.at[p, pl.ds(dim, dim)], osems[b]
            )
            return left, right

        for b in range(_NBUF):
            for c in gathers(b, b):
                c.start()

        def loop_body(g, carry):
            j0 = g * _NBUF
            for b in range(_NBUF):
                for c in gathers(j0 + b, b):
                    c.wait()
                for c in stores(j0 + b, b):
                    c.start()
            for b in range(_NBUF):
                for c in stores(j0 + b, b):
                    c.wait()
                nj = j0 + b + _NBUF

                @pl.when(nj < chunks_per_w)
                def _():
                    for c in gathers(nj, b):
                        c.start()

            return carry

        lax.fori_loop(0, n_groups, loop_body, 0)

    return run


def kernel(token_ids, weight):
    n_rows = token_ids.size
    dim = weight.shape[1]
    idx = (
        token_ids.reshape(n_rows // _CHUNK, _HALF, 2)
        .transpose(0, 2, 1)
        .reshape(n_rows // _CHUNK, _CHUNK)
        .astype(jnp.int32)
    )
    out = _build(n_rows, dim)(idx, weight)
    return out.reshape(token_ids.shape + (dim,))
